# baseline (speedup 1.0000x reference)
import functools

import jax
import jax.numpy as jnp
from jax import lax
from jax.experimental import pallas as pl
from jax.experimental.pallas import tpu as pltpu
from jax.experimental.pallas import tpu_sc as plsc

NUM_TABLES = 26
VOCAB = 100000
EMB = 16
B = 4096
NF = NUM_TABLES + 1

_NW = 32
_TOTAL_ROWS = B * NUM_TABLES
_ROWS_PER_W = _TOTAL_ROWS // _NW
_CHUNK = 128
_NCH = _ROWS_PER_W // _CHUNK

_RELAYOUT_CB = 20480


def _tc_relayout(tabT):
    n = tabT.shape[1]
    grid = (pl.cdiv(n, _RELAYOUT_CB),)

    def body(in_ref, out_ref):
        y = in_ref[...].T.astype(jnp.bfloat16)
        out_ref[...] = jnp.concatenate(
            [y, jnp.zeros((_RELAYOUT_CB, 128 - EMB), jnp.bfloat16)], axis=1)

    return pl.pallas_call(
        body,
        grid=grid,
        in_specs=[pl.BlockSpec((EMB, _RELAYOUT_CB), lambda i: (0, i))],
        out_specs=pl.BlockSpec((_RELAYOUT_CB, 128), lambda i: (i, 0)),
        out_shape=jax.ShapeDtypeStruct((n, 128), jnp.bfloat16),
    )(tabT)


def _sc_gather(tables_rm, idx3d):
    mesh = plsc.VectorSubcoreMesh(core_axis_name="c", subcore_axis_name="s")

    @functools.partial(
        pl.kernel,
        out_type=jax.ShapeDtypeStruct((_TOTAL_ROWS, EMB), jnp.bfloat16),
        mesh=mesh,
        scratch_types=[
            pltpu.VMEM((_NCH, _CHUNK), jnp.int32),
            pltpu.VMEM((_ROWS_PER_W, EMB), jnp.bfloat16),
            pltpu.SemaphoreType.DMA,
        ],
        compiler_params=pltpu.CompilerParams(use_tc_tiling_on_sc=False),
    )
    def k(tab_hbm, idx_hbm, out_hbm, idx_v, rows_v, sem):
        wid = lax.axis_index("s") * 2 + lax.axis_index("c")
        pltpu.sync_copy(idx_hbm.at[wid], idx_v)
        copies = []
        for j in range(_NCH):
            copies.append(
                pltpu.async_copy(
                    tab_hbm.at[idx_v.at[j]],
                    rows_v.at[pl.ds(j * _CHUNK, _CHUNK)],
                    sem,
                )
            )
        for c in copies:
            c.wait()
        pltpu.sync_copy(rows_v, out_hbm.at[pl.ds(wid * _ROWS_PER_W, _ROWS_PER_W)])

    return k(tables_rm, idx3d)


def _dense_body(dT, eT, wd0, b0, wd1, b1, wd2, b2, wo0, bo0, wo1, bo1, wf, bf,
                out_ref):
    x = jnp.maximum(jnp.dot(wd0[...], dT[...],
                            preferred_element_type=jnp.float32) + b0[...], 0.0)
    x = jnp.maximum(jnp.dot(wd1[...], x,
                            preferred_element_type=jnp.float32) + b1[...], 0.0)
    d2 = jnp.maximum(jnp.dot(wd2[...], x,
                             preferred_element_type=jnp.float32) + b2[...], 0.0)
    ct = jnp.concatenate([d2, eT[...].astype(jnp.float32)], axis=0)
    rows = [d2]
    for n in range(NF):
        cn = ct[n * EMB:(n + 1) * EMB]
        for m in range(n + 1, NF):
            rows.append(jnp.sum(cn * ct[m * EMB:(m + 1) * EMB], axis=0,
                                keepdims=True))
    cc = jnp.concatenate(rows, axis=0)
    h = jnp.maximum(jnp.dot(wo0[...], cc,
                            preferred_element_type=jnp.float32) + bo0[...], 0.0)
    h = jnp.maximum(jnp.dot(wo1[...], h,
                            preferred_element_type=jnp.float32) + bo1[...], 0.0)
    o = jnp.dot(wf[...], h, preferred_element_type=jnp.float32) + bf[...]
    out_ref[...] = jax.nn.sigmoid(o)


def _tc_dense(dT, eT, wd0, b0, wd1, b1, wd2, b2, wo0, bo0, wo1, bo1, wf, bf,
              block_b=512):
    grid = (B // block_b,)
    full = lambda shape: pl.BlockSpec(shape, lambda i: (0, 0))
    blk = lambda rows: pl.BlockSpec((rows, block_b), lambda i: (0, i))
    return pl.pallas_call(
        _dense_body,
        grid=grid,
        in_specs=[
            blk(13), blk(NUM_TABLES * EMB),
            full(wd0.shape), full(b0.shape),
            full(wd1.shape), full(b1.shape),
            full(wd2.shape), full(b2.shape),
            full(wo0.shape), full(bo0.shape),
            full(wo1.shape), full(bo1.shape),
            full(wf.shape), full(bf.shape),
        ],
        out_specs=blk(1),
        out_shape=jax.ShapeDtypeStruct((1, B), jnp.float32),
    )(dT, eT, wd0, b0, wd1, b1, wd2, b2, wo0, bo0, wo1, bo1, wf, bf)


def kernel(dense_features, tables, W_d0, b_d0, W_d1, b_d1, W_d2, b_d2,
           W_o0, b_o0, W_o1, b_o1, W_f, b_f, sparse_indices):
    offs = (jnp.arange(NUM_TABLES, dtype=sparse_indices.dtype) * VOCAB)[None, :]
    idx3d = ((sparse_indices + offs) * 8).reshape(_NW, _NCH, _CHUNK)
    n_rows = tables.shape[0]
    tables_rm = _tc_relayout(tables.T).reshape(n_rows * 8, EMB)
    emb_flat = _sc_gather(tables_rm, idx3d)
    eT = emb_flat.reshape(B, NUM_TABLES * EMB).T
    out = _tc_dense(
        dense_features.T, eT,
        W_d0.T, b_d0[:, None], W_d1.T, b_d1[:, None], W_d2.T, b_d2[:, None],
        W_o0.T, b_o0[:, None], W_o1.T, b_o1[:, None], W_f.T, b_f[:, None],
    )
    return out[0]


# speedup vs baseline: 4.6874x; 4.6874x over previous
import functools

import jax
import jax.numpy as jnp
from jax import lax
from jax.experimental import pallas as pl
from jax.experimental.pallas import tpu as pltpu
from jax.experimental.pallas import tpu_sc as plsc

NUM_TABLES = 26
VOCAB = 100000
EMB = 16
B = 4096
NF = NUM_TABLES + 1

_NW = 32
_TOTAL_ROWS = B * NUM_TABLES
_ROWS_PER_W = _TOTAL_ROWS // _NW
_CHUNK = 128
_NCH = _ROWS_PER_W // _CHUNK

_RELAYOUT_CB = 20480


def _tc_relayout(tabT):
    n = tabT.shape[1]
    grid = (pl.cdiv(n, _RELAYOUT_CB),)

    def body(in_ref, out_ref):
        y = in_ref[...].T
        out_ref[...] = jnp.concatenate(
            [y, jnp.zeros((_RELAYOUT_CB, 128 - EMB), jnp.float32)], axis=1)

    return pl.pallas_call(
        body,
        grid=grid,
        in_specs=[pl.BlockSpec((EMB, _RELAYOUT_CB), lambda i: (0, i))],
        out_specs=pl.BlockSpec((_RELAYOUT_CB, 128), lambda i: (i, 0)),
        out_shape=jax.ShapeDtypeStruct((n, 128), jnp.float32),
    )(tabT)


def _sc_gather(tables_rm, idx3d):
    mesh = plsc.VectorSubcoreMesh(core_axis_name="c", subcore_axis_name="s")

    @functools.partial(
        pl.kernel,
        out_type=jax.ShapeDtypeStruct((_TOTAL_ROWS, EMB), jnp.float32),
        mesh=mesh,
        scratch_types=[
            pltpu.VMEM((_NCH, _CHUNK), jnp.int32),
            pltpu.VMEM((_ROWS_PER_W, EMB), jnp.float32),
            pltpu.SemaphoreType.DMA,
        ],
        compiler_params=pltpu.CompilerParams(use_tc_tiling_on_sc=False),
    )
    def k(tab_hbm, idx_hbm, out_hbm, idx_v, rows_v, sem):
        wid = lax.axis_index("s") * 2 + lax.axis_index("c")
        pltpu.sync_copy(idx_hbm.at[wid], idx_v)
        copies = []
        for j in range(_NCH):
            copies.append(
                pltpu.async_copy(
                    tab_hbm.at[idx_v.at[j]],
                    rows_v.at[pl.ds(j * _CHUNK, _CHUNK)],
                    sem,
                )
            )
        for c in copies:
            c.wait()
        pltpu.sync_copy(rows_v, out_hbm.at[pl.ds(wid * _ROWS_PER_W, _ROWS_PER_W)])

    return k(tables_rm, idx3d)


def _dense_body(dT, eT, wd0, b0, wd1, b1, wd2, b2, wo0, bo0, wo1, bo1, wf, bf,
                out_ref):
    x = jnp.maximum(jnp.dot(wd0[...], dT[...],
                            preferred_element_type=jnp.float32) + b0[...], 0.0)
    x = jnp.maximum(jnp.dot(wd1[...], x,
                            preferred_element_type=jnp.float32) + b1[...], 0.0)
    d2 = jnp.maximum(jnp.dot(wd2[...], x,
                             preferred_element_type=jnp.float32) + b2[...], 0.0)
    ct = jnp.concatenate([d2, eT[...].astype(jnp.float32)], axis=0)
    rows = [d2]
    for n in range(NF):
        cn = ct[n * EMB:(n + 1) * EMB]
        for m in range(n + 1, NF):
            rows.append(jnp.sum(cn * ct[m * EMB:(m + 1) * EMB], axis=0,
                                keepdims=True))
    cc = jnp.concatenate(rows, axis=0)
    h = jnp.maximum(jnp.dot(wo0[...], cc,
                            preferred_element_type=jnp.float32) + bo0[...], 0.0)
    h = jnp.maximum(jnp.dot(wo1[...], h,
                            preferred_element_type=jnp.float32) + bo1[...], 0.0)
    o = jnp.dot(wf[...], h, preferred_element_type=jnp.float32) + bf[...]
    out_ref[...] = jax.nn.sigmoid(o)


def _tc_dense(dT, eT, wd0, b0, wd1, b1, wd2, b2, wo0, bo0, wo1, bo1, wf, bf,
              block_b=512):
    grid = (B // block_b,)
    full = lambda shape: pl.BlockSpec(shape, lambda i: (0, 0))
    blk = lambda rows: pl.BlockSpec((rows, block_b), lambda i: (0, i))
    return pl.pallas_call(
        _dense_body,
        grid=grid,
        in_specs=[
            blk(13), blk(NUM_TABLES * EMB),
            full(wd0.shape), full(b0.shape),
            full(wd1.shape), full(b1.shape),
            full(wd2.shape), full(b2.shape),
            full(wo0.shape), full(bo0.shape),
            full(wo1.shape), full(bo1.shape),
            full(wf.shape), full(bf.shape),
        ],
        out_specs=blk(1),
        out_shape=jax.ShapeDtypeStruct((1, B), jnp.float32),
    )(dT, eT, wd0, b0, wd1, b1, wd2, b2, wo0, bo0, wo1, bo1, wf, bf)


def kernel(dense_features, tables, W_d0, b_d0, W_d1, b_d1, W_d2, b_d2,
           W_o0, b_o0, W_o1, b_o1, W_f, b_f, sparse_indices):
    offs = (jnp.arange(NUM_TABLES, dtype=sparse_indices.dtype) * VOCAB)[None, :]
    idx3d = ((sparse_indices + offs) * 8).reshape(_NW, _NCH, _CHUNK)
    n_rows = tables.shape[0]
    tables_rm = _tc_relayout(tables.T).reshape(n_rows * 8, EMB)
    emb_flat = _sc_gather(tables_rm, idx3d)
    eT = emb_flat.reshape(B, NUM_TABLES * EMB).T
    out = _tc_dense(
        dense_features.T, eT,
        W_d0.T, b_d0[:, None], W_d1.T, b_d1[:, None], W_d2.T, b_d2[:, None],
        W_o0.T, b_o0[:, None], W_o1.T, b_o1[:, None], W_f.T, b_f[:, None],
    )
    return out[0]


# speedup vs baseline: 4.7657x; 1.0167x over previous
import functools

import jax
import jax.numpy as jnp
from jax import lax
from jax.experimental import pallas as pl
from jax.experimental.pallas import tpu as pltpu
from jax.experimental.pallas import tpu_sc as plsc

NUM_TABLES = 26
VOCAB = 100000
EMB = 16
B = 4096
NF = NUM_TABLES + 1

_NW = 32
_TOTAL_ROWS = B * NUM_TABLES
_ROWS_PER_W = _TOTAL_ROWS // _NW
_CHUNK = 128
_NCH = _ROWS_PER_W // _CHUNK

_RELAYOUT_CB = 40960


def _tc_relayout(tabT):
    n = tabT.shape[1]
    grid = (pl.cdiv(n, _RELAYOUT_CB),)

    def body(in_ref, out_ref):
        y = in_ref[...].T
        out_ref[...] = jnp.concatenate(
            [y, jnp.zeros((_RELAYOUT_CB, 128 - EMB), jnp.float32)], axis=1)

    return pl.pallas_call(
        body,
        grid=grid,
        in_specs=[pl.BlockSpec((EMB, _RELAYOUT_CB), lambda i: (0, i))],
        out_specs=pl.BlockSpec((_RELAYOUT_CB, 128), lambda i: (i, 0)),
        out_shape=jax.ShapeDtypeStruct((n, 128), jnp.float32),
    )(tabT)


def _sc_gather(tables_rm, idx3d):
    mesh = plsc.VectorSubcoreMesh(core_axis_name="c", subcore_axis_name="s")

    @functools.partial(
        pl.kernel,
        out_type=jax.ShapeDtypeStruct((_TOTAL_ROWS, EMB), jnp.float32),
        mesh=mesh,
        scratch_types=[
            pltpu.VMEM((_NCH, _CHUNK), jnp.int32),
            pltpu.VMEM((_ROWS_PER_W, EMB), jnp.float32),
            pltpu.SemaphoreType.DMA,
        ],
        compiler_params=pltpu.CompilerParams(use_tc_tiling_on_sc=False),
    )
    def k(tab_hbm, idx_hbm, out_hbm, idx_v, rows_v, sem):
        wid = lax.axis_index("s") * 2 + lax.axis_index("c")
        pltpu.sync_copy(idx_hbm.at[wid], idx_v)
        copies = []
        for j in range(_NCH):
            copies.append(
                pltpu.async_copy(
                    tab_hbm.at[idx_v.at[j]],
                    rows_v.at[pl.ds(j * _CHUNK, _CHUNK)],
                    sem,
                )
            )
        for c in copies:
            c.wait()
        pltpu.sync_copy(rows_v, out_hbm.at[pl.ds(wid * _ROWS_PER_W, _ROWS_PER_W)])

    return k(tables_rm, idx3d)


def _dense_body(dT, eT, wd0, b0, wd1, b1, wd2, b2, wo0, bo0, wo1, bo1, wf, bf,
                out_ref):
    x = jnp.maximum(jnp.dot(wd0[...], dT[...],
                            preferred_element_type=jnp.float32) + b0[...], 0.0)
    x = jnp.maximum(jnp.dot(wd1[...], x,
                            preferred_element_type=jnp.float32) + b1[...], 0.0)
    d2 = jnp.maximum(jnp.dot(wd2[...], x,
                             preferred_element_type=jnp.float32) + b2[...], 0.0)
    ct = jnp.concatenate([d2, eT[...].astype(jnp.float32)], axis=0)
    rows = [d2]
    for n in range(NF):
        cn = ct[n * EMB:(n + 1) * EMB]
        for m in range(n + 1, NF):
            rows.append(jnp.sum(cn * ct[m * EMB:(m + 1) * EMB], axis=0,
                                keepdims=True))
    cc = jnp.concatenate(rows, axis=0)
    h = jnp.maximum(jnp.dot(wo0[...], cc,
                            preferred_element_type=jnp.float32) + bo0[...], 0.0)
    h = jnp.maximum(jnp.dot(wo1[...], h,
                            preferred_element_type=jnp.float32) + bo1[...], 0.0)
    o = jnp.dot(wf[...], h, preferred_element_type=jnp.float32) + bf[...]
    out_ref[...] = jax.nn.sigmoid(o)


def _tc_dense(dT, eT, wd0, b0, wd1, b1, wd2, b2, wo0, bo0, wo1, bo1, wf, bf,
              block_b=512):
    grid = (B // block_b,)
    full = lambda shape: pl.BlockSpec(shape, lambda i: (0, 0))
    blk = lambda rows: pl.BlockSpec((rows, block_b), lambda i: (0, i))
    return pl.pallas_call(
        _dense_body,
        grid=grid,
        in_specs=[
            blk(13), blk(NUM_TABLES * EMB),
            full(wd0.shape), full(b0.shape),
            full(wd1.shape), full(b1.shape),
            full(wd2.shape), full(b2.shape),
            full(wo0.shape), full(bo0.shape),
            full(wo1.shape), full(bo1.shape),
            full(wf.shape), full(bf.shape),
        ],
        out_specs=blk(1),
        out_shape=jax.ShapeDtypeStruct((1, B), jnp.float32),
    )(dT, eT, wd0, b0, wd1, b1, wd2, b2, wo0, bo0, wo1, bo1, wf, bf)


def kernel(dense_features, tables, W_d0, b_d0, W_d1, b_d1, W_d2, b_d2,
           W_o0, b_o0, W_o1, b_o1, W_f, b_f, sparse_indices):
    offs = (jnp.arange(NUM_TABLES, dtype=sparse_indices.dtype) * VOCAB)[None, :]
    idx3d = ((sparse_indices + offs) * 8).reshape(_NW, _NCH, _CHUNK)
    n_rows = tables.shape[0]
    tables_rm = _tc_relayout(tables.T).reshape(n_rows * 8, EMB)
    emb_flat = _sc_gather(tables_rm, idx3d)
    eT = emb_flat.reshape(B, NUM_TABLES * EMB).T
    out = _tc_dense(
        dense_features.T, eT,
        W_d0.T, b_d0[:, None], W_d1.T, b_d1[:, None], W_d2.T, b_d2[:, None],
        W_o0.T, b_o0[:, None], W_o1.T, b_o1[:, None], W_f.T, b_f[:, None],
    )
    return out[0]


# speedup vs baseline: 4.7801x; 1.0030x over previous
import functools

import jax
import jax.numpy as jnp
from jax import lax
from jax.experimental import pallas as pl
from jax.experimental.pallas import tpu as pltpu
from jax.experimental.pallas import tpu_sc as plsc

NUM_TABLES = 26
VOCAB = 100000
EMB = 16
B = 4096
NF = NUM_TABLES + 1

_NW = 32
_TOTAL_ROWS = B * NUM_TABLES
_ROWS_PER_W = _TOTAL_ROWS // _NW
_CHUNK = 128
_NCH = _ROWS_PER_W // _CHUNK

_RELAYOUT_CB = 53248


def _tc_relayout(tabT):
    n = tabT.shape[1]
    grid = (pl.cdiv(n, _RELAYOUT_CB),)

    def body(in_ref, out_ref):
        y = in_ref[...].T
        out_ref[...] = jnp.concatenate(
            [y, jnp.zeros((_RELAYOUT_CB, 128 - EMB), jnp.float32)], axis=1)

    return pl.pallas_call(
        body,
        grid=grid,
        in_specs=[pl.BlockSpec((EMB, _RELAYOUT_CB), lambda i: (0, i))],
        out_specs=pl.BlockSpec((_RELAYOUT_CB, 128), lambda i: (i, 0)),
        out_shape=jax.ShapeDtypeStruct((n, 128), jnp.float32),
    )(tabT)


def _sc_gather(tables_rm, idx3d):
    mesh = plsc.VectorSubcoreMesh(core_axis_name="c", subcore_axis_name="s")

    @functools.partial(
        pl.kernel,
        out_type=jax.ShapeDtypeStruct((_TOTAL_ROWS, EMB), jnp.float32),
        mesh=mesh,
        scratch_types=[
            pltpu.VMEM((_NCH, _CHUNK), jnp.int32),
            pltpu.VMEM((_ROWS_PER_W, EMB), jnp.float32),
            pltpu.SemaphoreType.DMA,
        ],
        compiler_params=pltpu.CompilerParams(use_tc_tiling_on_sc=False),
    )
    def k(tab_hbm, idx_hbm, out_hbm, idx_v, rows_v, sem):
        wid = lax.axis_index("s") * 2 + lax.axis_index("c")
        pltpu.sync_copy(idx_hbm.at[wid], idx_v)
        copies = []
        for j in range(_NCH):
            copies.append(
                pltpu.async_copy(
                    tab_hbm.at[idx_v.at[j]],
                    rows_v.at[pl.ds(j * _CHUNK, _CHUNK)],
                    sem,
                )
            )
        for c in copies:
            c.wait()
        pltpu.sync_copy(rows_v, out_hbm.at[pl.ds(wid * _ROWS_PER_W, _ROWS_PER_W)])

    return k(tables_rm, idx3d)


def _dense_body(dT, eT, wd0, b0, wd1, b1, wd2, b2, wo0, bo0, wo1, bo1, wf, bf,
                out_ref):
    x = jnp.maximum(jnp.dot(wd0[...], dT[...],
                            preferred_element_type=jnp.float32) + b0[...], 0.0)
    x = jnp.maximum(jnp.dot(wd1[...], x,
                            preferred_element_type=jnp.float32) + b1[...], 0.0)
    d2 = jnp.maximum(jnp.dot(wd2[...], x,
                             preferred_element_type=jnp.float32) + b2[...], 0.0)
    ct = jnp.concatenate([d2, eT[...].astype(jnp.float32)], axis=0)
    rows = [d2]
    for n in range(NF):
        cn = ct[n * EMB:(n + 1) * EMB]
        for m in range(n + 1, NF):
            rows.append(jnp.sum(cn * ct[m * EMB:(m + 1) * EMB], axis=0,
                                keepdims=True))
    cc = jnp.concatenate(rows, axis=0)
    h = jnp.maximum(jnp.dot(wo0[...], cc,
                            preferred_element_type=jnp.float32) + bo0[...], 0.0)
    h = jnp.maximum(jnp.dot(wo1[...], h,
                            preferred_element_type=jnp.float32) + bo1[...], 0.0)
    o = jnp.dot(wf[...], h, preferred_element_type=jnp.float32) + bf[...]
    out_ref[...] = jax.nn.sigmoid(o)


def _tc_dense(dT, eT, wd0, b0, wd1, b1, wd2, b2, wo0, bo0, wo1, bo1, wf, bf,
              block_b=512):
    grid = (B // block_b,)
    full = lambda shape: pl.BlockSpec(shape, lambda i: (0, 0))
    blk = lambda rows: pl.BlockSpec((rows, block_b), lambda i: (0, i))
    return pl.pallas_call(
        _dense_body,
        grid=grid,
        in_specs=[
            blk(13), blk(NUM_TABLES * EMB),
            full(wd0.shape), full(b0.shape),
            full(wd1.shape), full(b1.shape),
            full(wd2.shape), full(b2.shape),
            full(wo0.shape), full(bo0.shape),
            full(wo1.shape), full(bo1.shape),
            full(wf.shape), full(bf.shape),
        ],
        out_specs=blk(1),
        out_shape=jax.ShapeDtypeStruct((1, B), jnp.float32),
    )(dT, eT, wd0, b0, wd1, b1, wd2, b2, wo0, bo0, wo1, bo1, wf, bf)


def kernel(dense_features, tables, W_d0, b_d0, W_d1, b_d1, W_d2, b_d2,
           W_o0, b_o0, W_o1, b_o1, W_f, b_f, sparse_indices):
    offs = (jnp.arange(NUM_TABLES, dtype=sparse_indices.dtype) * VOCAB)[None, :]
    idx3d = ((sparse_indices + offs) * 8).reshape(_NW, _NCH, _CHUNK)
    n_rows = tables.shape[0]
    tables_rm = _tc_relayout(tables.T).reshape(n_rows * 8, EMB)
    emb_flat = _sc_gather(tables_rm, idx3d)
    eT = emb_flat.reshape(B, NUM_TABLES * EMB).T
    out = _tc_dense(
        dense_features.T, eT,
        W_d0.T, b_d0[:, None], W_d1.T, b_d1[:, None], W_d2.T, b_d2[:, None],
        W_o0.T, b_o0[:, None], W_o1.T, b_o1[:, None], W_f.T, b_f[:, None],
    )
    return out[0]
